# Initial kernel scaffold; baseline (speedup 1.0000x reference)
import jax
import jax.numpy as jnp
from jax.experimental import pallas as pl


def kernel(inputs, codebooks, train):
    raise NotImplementedError("write your pallas kernel here")


# speedup vs baseline: 1.3653x; 1.3653x over previous
import functools

import jax
import jax.numpy as jnp
from jax import lax
from jax.experimental import pallas as pl
from jax.experimental.pallas import tpu as pltpu
from jax.experimental.pallas import tpu_sc as plsc

NS = 4
NC = 1024
SD = 128
TOKENS = 9216
TBLK = 512

_NUM_CORES = 2
_NUM_SUBCORES = 16
_NW = _NUM_CORES * _NUM_SUBCORES
_B = TOKENS * NS
_B_PER_W = _B // _NW
_CHUNK = 576


def _dist_body(x_ref, cb_ref, nn_ref, loss_ref):
    x = x_ref[...]
    acc = jnp.zeros((TBLK,), jnp.float32)
    for s in range(NS):
        xs = x[:, s * SD:(s + 1) * SD]
        cbs = cb_ref[s]
        mm = lax.dot_general(xs, cbs, (((1,), (1,)), ((), ())),
                             preferred_element_type=jnp.float32)
        xn = jnp.sum(xs * xs, axis=1)
        cn = jnp.sum(cbs * cbs, axis=1)
        d = (xn[:, None] - 2.0 * mm) + cn[None, :]
        m = jnp.min(d, axis=1)
        iota = lax.broadcasted_iota(jnp.int32, d.shape, 1)
        idx = jnp.min(jnp.where(d == m[:, None], iota, jnp.int32(NC)), axis=1)
        nn_ref[s, :] = idx
        acc = acc + m
    loss_ref[0, :] = acc * (1.0 / (NS * SD))


def _distances(flat, codebooks):
    grid = (TOKENS // TBLK,)
    return pl.pallas_call(
        _dist_body,
        grid=grid,
        in_specs=[
            pl.BlockSpec((TBLK, NS * SD), lambda i: (i, 0)),
            pl.BlockSpec((NS, NC, SD), lambda i: (0, 0, 0)),
        ],
        out_specs=[
            pl.BlockSpec((NS, TBLK), lambda i: (0, i)),
            pl.BlockSpec((1, TBLK), lambda i: (0, i)),
        ],
        out_shape=[
            jax.ShapeDtypeStruct((NS, TOKENS), jnp.int32),
            jax.ShapeDtypeStruct((1, TOKENS), jnp.float32),
        ],
    )(flat, codebooks)


@functools.cache
def _make_sc_gather():
    mesh = plsc.VectorSubcoreMesh(core_axis_name="c", subcore_axis_name="s")

    @functools.partial(
        pl.kernel,
        mesh=mesh,
        out_type=jax.ShapeDtypeStruct((_B, SD), jnp.float32),
        scratch_types=[
            pltpu.VMEM((_B_PER_W,), jnp.int32),
            pltpu.VMEM((_CHUNK, SD), jnp.float32),
            pltpu.SemaphoreType.DMA,
        ],
    )
    def _sc_gather(table_hbm, idx_hbm, out_hbm, idx_v, rows_v, sem):
        wid = lax.axis_index("s") * _NUM_CORES + lax.axis_index("c")
        base = wid * _B_PER_W
        pltpu.sync_copy(idx_hbm.at[pl.ds(base, _B_PER_W)], idx_v)
        for c in range(_B_PER_W // _CHUNK):
            pltpu.async_copy(
                table_hbm.at[idx_v.at[pl.ds(c * _CHUNK, _CHUNK)]], rows_v, sem
            ).wait()
            pltpu.sync_copy(rows_v, out_hbm.at[pl.ds(base + c * _CHUNK, _CHUNK)])

    return _sc_gather


def kernel(inputs, codebooks, train):
    flat = jnp.reshape(inputs, (-1, NS * SD))
    nn, loss = _distances(flat, codebooks)
    offs = (jnp.arange(NS, dtype=jnp.int32) * NC)[:, None]
    nn_global = jnp.reshape(jnp.transpose(nn + offs), (_B,))
    table = jnp.reshape(codebooks, (NS * NC, SD))
    gathered = _make_sc_gather()(table, nn_global)
    quantized = jnp.reshape(gathered, inputs.shape)
    qloss = jnp.reshape(loss, inputs.shape[:-1] + (1,))
    nn_out = jnp.reshape(nn, (NS,) + inputs.shape[:-1])
    codebook = jnp.reshape(codebooks, (NS * NC, SD))
    return quantized, qloss, nn_out, codebook
